# baseline (speedup 1.0000x reference)
import math

import jax
import jax.numpy as jnp
from jax.experimental import pallas as pl
from jax.experimental.pallas import tpu as pltpu

_B, _S, _V = 64, 4, 100000
_R = _B * _S
_LS = 0.1
_CONF = 1.0 - _LS
_SV = _LS / (_V - 2)
_ENT = (_V - 2) * _SV * math.log(_SV) + _CONF * math.log(_CONF)

_VB = 1024
_NBLK = (_V + _VB - 1) // _VB


def _loss_kernel(t_ref, x_ref, o_ref, acc_ref):
    j = pl.program_id(0)
    t = t_ref[:, :]
    x = x_ref[:, :]
    col = jax.lax.broadcasted_iota(jnp.int32, (_R, _VB), 1) + j * _VB
    zero = (t == 0) | (col == 0) | (col >= _V)
    coef = jnp.where(col == t, -_CONF, -_SV)
    val = jnp.where(zero, 0.0, coef * x)

    @pl.when(j == 0)
    def _():
        acc_ref[...] = val

    @pl.when(j > 0)
    def _():
        acc_ref[...] = acc_ref[...] + val

    @pl.when(j == _NBLK - 1)
    def _():
        n_active = jnp.sum(jnp.where(t == 0, 0.0, 1.0))
        o_ref[0, 0] = jnp.sum(acc_ref[...]) + jnp.float32(_ENT) * n_active


def kernel(output, target, one_hot):
    del one_hot
    x = output.reshape(_R, _V)
    t = target.reshape(_R, 1)
    out = pl.pallas_call(
        _loss_kernel,
        grid=(_NBLK,),
        in_specs=[
            pl.BlockSpec((_R, 1), lambda j: (0, 0)),
            pl.BlockSpec((_R, _VB), lambda j: (0, j)),
        ],
        out_specs=pl.BlockSpec(memory_space=pltpu.SMEM),
        out_shape=jax.ShapeDtypeStruct((1, 1), jnp.float32),
        scratch_shapes=[pltpu.VMEM((_R, _VB), jnp.float32)],
        compiler_params=pltpu.CompilerParams(
            dimension_semantics=("arbitrary",),
        ),
    )(t, x)
    return out[0, 0]
